# baseline (device time: 28987 ns/iter reference)
import jax
import jax.numpy as jnp
from jax import lax
from jax.experimental import pallas as pl
from jax.experimental.pallas import tpu as pltpu

N_RING = 4
N_HOPS = N_RING - 1
Q = 4


def kernel(x, dy):
    k_per, d = x.shape
    _, f = dy.shape
    m_per = d // N_RING
    rows = m_per // Q

    def body(x_hbm, dy_hbm, out_hbm, xv_ref, dyv_ref, p_ref, send_ref,
             recv_ref, o_ref, in_sems, out_sems, send_sems, recv_sems):
        my_x = lax.axis_index("x")
        my_y = lax.axis_index("y")
        my_z = lax.axis_index("z")
        left = lax.rem(my_z + N_RING - 1, N_RING)
        right = lax.rem(my_z + 1, N_RING)

        x_copy = pltpu.make_async_copy(x_hbm, xv_ref, in_sems.at[0])
        dy_copy = pltpu.make_async_copy(dy_hbm, dyv_ref, in_sems.at[1])
        x_copy.start()
        dy_copy.start()

        barrier_sem = pltpu.get_barrier_semaphore()
        for nbr in (left, right):
            pl.semaphore_signal(
                barrier_sem, inc=1,
                device_id=(my_x, my_y, nbr),
                device_id_type=pl.DeviceIdType.MESH,
            )
        pl.semaphore_wait(barrier_sem, 2)

        x_copy.wait()
        dy_copy.wait()

        p_ref[:, :] = lax.dot_general(
            xv_ref[:, :].astype(jnp.bfloat16),
            dyv_ref[:, :].astype(jnp.bfloat16),
            dimension_numbers=(((0,), (0,)), ((), ())),
            preferred_element_type=jnp.float32,
        ).astype(jnp.bfloat16)

        def p_slice(j, q):
            c = lax.rem(my_z - 1 - j + 2 * N_RING, N_RING)
            return p_ref[pl.ds(c * m_per + q * rows, rows), :]

        def rdma_sq(s, q):
            sl = pl.ds(q * rows, rows)
            return pltpu.make_async_remote_copy(
                src_ref=send_ref.at[s, sl],
                dst_ref=recv_ref.at[s, sl],
                send_sem=send_sems.at[s * Q + q],
                recv_sem=recv_sems.at[s * Q + q],
                device_id=(my_x, my_y, right),
                device_id_type=pl.DeviceIdType.MESH,
            )

        for q in range(Q):
            send_ref[0, pl.ds(q * rows, rows)] = p_slice(0, q)
            rdma_sq(0, q).start()

        for s in range(1, N_HOPS):
            for q in range(Q):
                sl = pl.ds(q * rows, rows)
                rdma_sq(s - 1, q).wait_recv()
                send_ref[s, sl] = (
                    recv_ref[s - 1, sl].astype(jnp.float32)
                    + p_slice(s, q).astype(jnp.float32)
                ).astype(jnp.bfloat16)
                rdma_sq(s, q).start()

        out_copies = []
        for q in range(Q):
            sl = pl.ds(q * rows, rows)
            rdma_sq(N_HOPS - 1, q).wait_recv()
            o_ref[sl, :] = (
                recv_ref[N_HOPS - 1, sl].astype(jnp.float32)
                + p_slice(N_HOPS, q).astype(jnp.float32)
            )
            oc = pltpu.make_async_copy(
                o_ref.at[sl], out_hbm.at[sl], out_sems.at[q]
            )
            oc.start()
            out_copies.append(oc)

        for oc in out_copies:
            oc.wait()
        for s in range(N_HOPS):
            for q in range(Q):
                rdma_sq(s, q).wait_send()

    return pl.pallas_call(
        body,
        out_shape=jax.ShapeDtypeStruct((m_per, f), jnp.float32),
        in_specs=[
            pl.BlockSpec(memory_space=pl.ANY),
            pl.BlockSpec(memory_space=pl.ANY),
        ],
        out_specs=pl.BlockSpec(memory_space=pl.ANY),
        scratch_shapes=[
            pltpu.VMEM((k_per, d), jnp.float32),
            pltpu.VMEM((k_per, f), jnp.float32),
            pltpu.VMEM((d, f), jnp.bfloat16),
            pltpu.VMEM((N_HOPS, m_per, f), jnp.bfloat16),
            pltpu.VMEM((N_HOPS, m_per, f), jnp.bfloat16),
            pltpu.VMEM((m_per, f), jnp.float32),
            pltpu.SemaphoreType.DMA((2,)),
            pltpu.SemaphoreType.DMA((Q,)),
            pltpu.SemaphoreType.DMA((N_HOPS * Q,)),
            pltpu.SemaphoreType.DMA((N_HOPS * Q,)),
        ],
        compiler_params=pltpu.CompilerParams(collective_id=0),
    )(x, dy)


# device time: 28472 ns/iter; 1.0181x vs baseline; 1.0181x over previous
import jax
import jax.numpy as jnp
from jax import lax
from jax.experimental import pallas as pl
from jax.experimental.pallas import tpu as pltpu

N_RING = 4
N_HOPS = N_RING - 1
Q = 4


def kernel(x, dy):
    k_per, d = x.shape
    _, f = dy.shape
    m_per = d // N_RING
    rows = m_per // Q

    def body(x_ref, dy_ref, out_ref, p_ref, send_ref, recv_ref,
             send_sems, recv_sems):
        my_x = lax.axis_index("x")
        my_y = lax.axis_index("y")
        my_z = lax.axis_index("z")
        left = lax.rem(my_z + N_RING - 1, N_RING)
        right = lax.rem(my_z + 1, N_RING)

        barrier_sem = pltpu.get_barrier_semaphore()
        for nbr in (left, right):
            pl.semaphore_signal(
                barrier_sem, inc=1,
                device_id=(my_x, my_y, nbr),
                device_id_type=pl.DeviceIdType.MESH,
            )
        pl.semaphore_wait(barrier_sem, 2)

        p_ref[:, :] = lax.dot_general(
            x_ref[:, :].astype(jnp.bfloat16),
            dy_ref[:, :].astype(jnp.bfloat16),
            dimension_numbers=(((0,), (0,)), ((), ())),
            preferred_element_type=jnp.float32,
        )

        def p_slice(j, q):
            c = lax.rem(my_z - 1 - j + 2 * N_RING, N_RING)
            return p_ref[pl.ds(c * m_per + q * rows, rows), :]

        def rdma_sq(s, q):
            sl = pl.ds(q * rows, rows)
            return pltpu.make_async_remote_copy(
                src_ref=send_ref.at[s, sl],
                dst_ref=recv_ref.at[s, sl],
                send_sem=send_sems.at[s * Q + q],
                recv_sem=recv_sems.at[s * Q + q],
                device_id=(my_x, my_y, right),
                device_id_type=pl.DeviceIdType.MESH,
            )

        for q in range(Q):
            send_ref[0, pl.ds(q * rows, rows)] = p_slice(0, q).astype(
                jnp.bfloat16
            )
            rdma_sq(0, q).start()

        for s in range(1, N_HOPS):
            for q in range(Q):
                sl = pl.ds(q * rows, rows)
                rdma_sq(s - 1, q).wait_recv()
                send_ref[s, sl] = (
                    recv_ref[s - 1, sl].astype(jnp.float32) + p_slice(s, q)
                ).astype(jnp.bfloat16)
                rdma_sq(s, q).start()

        for q in range(Q):
            sl = pl.ds(q * rows, rows)
            rdma_sq(N_HOPS - 1, q).wait_recv()
            out_ref[sl, :] = (
                recv_ref[N_HOPS - 1, sl].astype(jnp.float32)
                + p_slice(N_HOPS, q)
            )

        for s in range(N_HOPS):
            for q in range(Q):
                rdma_sq(s, q).wait_send()

    return pl.pallas_call(
        body,
        out_shape=jax.ShapeDtypeStruct((m_per, f), jnp.float32),
        in_specs=[
            pl.BlockSpec(memory_space=pltpu.VMEM),
            pl.BlockSpec(memory_space=pltpu.VMEM),
        ],
        out_specs=pl.BlockSpec(memory_space=pltpu.VMEM),
        scratch_shapes=[
            pltpu.VMEM((d, f), jnp.float32),
            pltpu.VMEM((N_HOPS, m_per, f), jnp.bfloat16),
            pltpu.VMEM((N_HOPS, m_per, f), jnp.bfloat16),
            pltpu.SemaphoreType.DMA((N_HOPS * Q,)),
            pltpu.SemaphoreType.DMA((N_HOPS * Q,)),
        ],
        compiler_params=pltpu.CompilerParams(collective_id=0),
    )(x, dy)
